# baseline (speedup 1.0000x reference)
import jax
import jax.numpy as jnp
from jax.experimental import pallas as pl
from jax.experimental.pallas import tpu as pltpu

R = 512
S = 128
N_ROWS = 8192
N_COLS = 2048


def _body(a_ref, o_ref, carry_ref):
    i = pl.program_id(0)

    @pl.when(i == 0)
    def _():
        carry_ref[...] = jnp.zeros_like(carry_ref)

    rows = jax.lax.broadcasted_iota(jnp.int32, (S, S), 0)
    cols = jax.lax.broadcasted_iota(jnp.int32, (S, S), 1)
    strict_lower = (cols < rows).astype(jnp.bfloat16)

    carry = carry_ref[...]
    for k in range(R // S):
        sub = a_ref[pl.ds(k * S, S), :]
        local_ex = jnp.dot(strict_lower, sub.astype(jnp.bfloat16),
                           preferred_element_type=jnp.float32)
        out = local_ex + carry
        o_ref[pl.ds(k * S, S), :] = out
        carry = carry + jnp.sum(sub, axis=0, keepdims=True)
    carry_ref[...] = carry


@jax.jit
def kernel(a):
    n_chunks = N_ROWS // R
    return pl.pallas_call(
        _body,
        grid=(n_chunks,),
        in_specs=[pl.BlockSpec((R, N_COLS), lambda i: (i, 0))],
        out_specs=pl.BlockSpec((R, N_COLS), lambda i: (i, 0)),
        out_shape=jax.ShapeDtypeStruct((N_ROWS, N_COLS), jnp.float32),
        scratch_shapes=[pltpu.VMEM((1, N_COLS), jnp.float32)],
        compiler_params=pltpu.CompilerParams(
            dimension_semantics=("arbitrary",),
        ),
    )(a)
